# speedup vs baseline: 1.1950x; 1.1950x over previous
import jax
import jax.numpy as jnp
from jax import lax
from jax.experimental import pallas as pl
from jax.experimental.pallas import tpu as pltpu
from jax.experimental.pallas import tpu_sc as plsc

DIM = 768
NC = 2
NS = 16
NW = NC * NS
K = 80


def _gather_body(ctx_idx, q_idx, table, ctx_out, q_out, idx_v, rows_v, sem):
    wid = lax.axis_index("s") * NC + lax.axis_index("c")
    ctx_per_w = ctx_idx.shape[0] // NW
    q_per_w = q_idx.shape[0] // NW

    def run(idx_hbm, out_hbm, per_w):
        base = wid * per_w

        @pl.loop(0, per_w // K)
        def _chunk(i):
            off = base + i * K
            pltpu.sync_copy(idx_hbm.at[pl.ds(off, K)], idx_v)
            pltpu.async_copy(table.at[idx_v], rows_v, sem).wait()
            pltpu.sync_copy(rows_v, out_hbm.at[pl.ds(off, K)])

    run(ctx_idx, ctx_out, ctx_per_w)
    run(q_idx, q_out, q_per_w)


@jax.jit
def _gather(ctx_idx, q_idx, table):
    n_ctx = ctx_idx.shape[0]
    n_q = q_idx.shape[0]
    mesh = plsc.VectorSubcoreMesh(core_axis_name="c", subcore_axis_name="s")
    f = pl.kernel(
        _gather_body,
        out_type=(
            jax.ShapeDtypeStruct((n_ctx, DIM), jnp.float32),
            jax.ShapeDtypeStruct((n_q, DIM), jnp.float32),
        ),
        mesh=mesh,
        scratch_types=[
            pltpu.VMEM((K,), jnp.int32),
            pltpu.VMEM((K, DIM), jnp.float32),
            pltpu.SemaphoreType.DMA,
        ],
    )
    return f(ctx_idx, q_idx, table)


def kernel(context, context_lengths, question, question_lengths, table):
    ctx_idx = context.reshape(-1).astype(jnp.int32)
    q_idx = question.reshape(-1).astype(jnp.int32)
    ctx_e, q_e = _gather(ctx_idx, q_idx, table)
    ctx_e = ctx_e.reshape(context.shape + (DIM,))
    q_e = q_e.reshape(question.shape + (DIM,))
    return (ctx_e, ctx_e, q_e)


# speedup vs baseline: 1.2377x; 1.0357x over previous
import jax
import jax.numpy as jnp
from jax import lax
from jax.experimental import pallas as pl
from jax.experimental.pallas import tpu as pltpu
from jax.experimental.pallas import tpu_sc as plsc

DIM = 768
NC = 2
NS = 16
NW = NC * NS
K = 80


def _gather_body(ctx_idx, q_idx, table, ctx_out, q_out,
                 idx_v, buf0, buf1, sem0, sem1):
    wid = lax.axis_index("s") * NC + lax.axis_index("c")
    ctx_per_w = ctx_idx.shape[0] // NW
    q_per_w = q_idx.shape[0] // NW
    n_ctx_chunks = ctx_per_w // K
    n_chunks = (ctx_per_w + q_per_w) // K

    pltpu.sync_copy(ctx_idx.at[pl.ds(wid * ctx_per_w, ctx_per_w)],
                    idx_v.at[pl.ds(0, ctx_per_w)])
    pltpu.sync_copy(q_idx.at[pl.ds(wid * q_per_w, q_per_w)],
                    idx_v.at[pl.ds(ctx_per_w, q_per_w)])

    def start_gather(c, buf, sem):
        pltpu.make_async_copy(
            table.at[idx_v.at[pl.ds(c * K, K)]], buf, sem).start()

    def wait_gather(buf, sem):
        pltpu.make_async_copy(
            table.at[idx_v.at[pl.ds(0, K)]], buf, sem).wait()

    def store_chunk(c, buf):
        @pl.when(c < n_ctx_chunks)
        def _():
            pltpu.sync_copy(
                buf, ctx_out.at[pl.ds(wid * ctx_per_w + c * K, K)])

        @pl.when(c >= n_ctx_chunks)
        def _():
            pltpu.sync_copy(
                buf, q_out.at[pl.ds(wid * q_per_w + (c - n_ctx_chunks) * K, K)])

    start_gather(0, buf0, sem0)
    start_gather(1, buf1, sem1)

    @pl.loop(0, n_chunks // 2)
    def _pair(j):
        for buf, sem, par in ((buf0, sem0, 0), (buf1, sem1, 1)):
            c = 2 * j + par
            wait_gather(buf, sem)
            store_chunk(c, buf)

            @pl.when(c + 2 < n_chunks)
            def _():
                start_gather(c + 2, buf, sem)


@jax.jit
def _gather(ctx_idx, q_idx, table):
    n_ctx = ctx_idx.shape[0]
    n_q = q_idx.shape[0]
    per_w = (n_ctx + n_q) // NW
    mesh = plsc.VectorSubcoreMesh(core_axis_name="c", subcore_axis_name="s")
    f = pl.kernel(
        _gather_body,
        out_type=(
            jax.ShapeDtypeStruct((n_ctx, DIM), jnp.float32),
            jax.ShapeDtypeStruct((n_q, DIM), jnp.float32),
        ),
        mesh=mesh,
        scratch_types=[
            pltpu.VMEM((per_w,), jnp.int32),
            pltpu.VMEM((K, DIM), jnp.float32),
            pltpu.VMEM((K, DIM), jnp.float32),
            pltpu.SemaphoreType.DMA,
            pltpu.SemaphoreType.DMA,
        ],
    )
    return f(ctx_idx, q_idx, table)


def kernel(context, context_lengths, question, question_lengths, table):
    ctx_idx = context.reshape(-1).astype(jnp.int32)
    q_idx = question.reshape(-1).astype(jnp.int32)
    ctx_e, q_e = _gather(ctx_idx, q_idx, table)
    ctx_e = ctx_e.reshape(context.shape + (DIM,))
    q_e = q_e.reshape(question.shape + (DIM,))
    return (ctx_e, ctx_e, q_e)
